# Initial kernel scaffold; baseline (speedup 1.0000x reference)
import jax
import jax.numpy as jnp
from jax.experimental import pallas as pl


def kernel(x, table):
    raise NotImplementedError("write your pallas kernel here")


# speedup vs baseline: 1.2958x; 1.2958x over previous
import functools
import math

import jax
import jax.numpy as jnp
from jax import lax
from jax.experimental import pallas as pl
from jax.experimental.pallas import tpu as pltpu
from jax.experimental.pallas import tpu_sc as plsc

NC = 2
NS = 16
NW = NC * NS
L = 16

D_MODEL = 1024
SCALE = math.sqrt(D_MODEL)


@functools.partial(jax.jit, static_argnums=(2, 3))
def _gather_scale(table, idx, B, D):
    b_per_w = B // NW
    CHUNK = 32
    NCHUNK = b_per_w // CHUNK
    mesh = plsc.VectorSubcoreMesh(core_axis_name="c", subcore_axis_name="s")

    @functools.partial(
        pl.kernel,
        out_type=jax.ShapeDtypeStruct((B, D), jnp.float32),
        mesh=mesh,
        scratch_types=[
            pltpu.VMEM((b_per_w,), jnp.int32),
            pltpu.VMEM((CHUNK, D), jnp.float32),
            pltpu.VMEM((CHUNK, D), jnp.float32),
            pltpu.SemaphoreType.DMA,
            pltpu.SemaphoreType.DMA,
            pltpu.SemaphoreType.DMA,
            pltpu.SemaphoreType.DMA,
        ],
    )
    def body(table_hbm, idx_hbm, out_hbm, idx_v, buf0, buf1, g0, g1, o0, o1):
        wid = lax.axis_index("s") * NC + lax.axis_index("c")
        base = wid * b_per_w
        pltpu.sync_copy(idx_hbm.at[pl.ds(base, b_per_w)], idx_v)

        bufs = (buf0, buf1)
        gsems = (g0, g1)
        osems = (o0, o1)

        def start_gather(c):
            b = c % 2
            return pltpu.async_copy(
                table_hbm.at[idx_v.at[pl.ds(c * CHUNK, CHUNK)]],
                bufs[b], gsems[b])

        def scale_buf(buf):
            def row(r, _):
                for j in range(D // L):
                    sl = (r, pl.ds(j * L, L))
                    buf[sl] = buf[sl] * SCALE
                return ()
            lax.fori_loop(0, CHUNK, row, ())

        gd = [None, None]
        od = [None, None]
        gd[0] = start_gather(0)
        for c in range(NCHUNK):
            b = c % 2
            if c + 1 < NCHUNK:
                if od[1 - b] is not None:
                    od[1 - b].wait()
                gd[1 - b] = start_gather(c + 1)
            gd[b].wait()
            scale_buf(bufs[b])
            od[b] = pltpu.async_copy(
                bufs[b], out_hbm.at[pl.ds(base + c * CHUNK, CHUNK)], osems[b])
        for d in od:
            if d is not None:
                d.wait()

    return body(table, idx)


def kernel(x, table):
    B = x.shape[0] * x.shape[1]
    D = table.shape[1]
    idx = x.reshape(B).astype(jnp.int32)
    out = _gather_scale(table, idx, B, D)
    return out.reshape(x.shape + (D,))
